# Initial kernel scaffold; baseline (speedup 1.0000x reference)
import jax
import jax.numpy as jnp
from jax.experimental import pallas as pl


def kernel(hidden_states, Wq, Wk, Wv, Wo, v_emb0, v_emb1):
    raise NotImplementedError("write your pallas kernel here")


# speedup vs baseline: 15.5079x; 15.5079x over previous
import functools
import math

import jax
import jax.numpy as jnp
import numpy as np
from jax.experimental import pallas as pl
from jax.experimental.pallas import tpu as pltpu

H = 96
QK = 8
VB = 8
W = 8
T = 512

_INV_SQRT_QK = 1.0 / math.sqrt(float(QK))
_STD_OF_BM = np.arange(H * QK).reshape(H, QK).T.reshape(-1)


def _rosa_body(h_ref, wq_ref, wk_ref, wv_ref, wo_ref, bias_ref, out_ref,
               kt_ref, vt_ref):
    i = pl.program_id(0)
    h = h_ref[...]
    qb = jnp.tanh(jnp.dot(h, wq_ref[...], preferred_element_type=jnp.float32))
    kb = jnp.tanh(jnp.dot(h, wk_ref[...], preferred_element_type=jnp.float32))
    vb = jax.nn.sigmoid(jnp.dot(h, wv_ref[...], preferred_element_type=jnp.float32))

    first = i == 0
    prev_k = jnp.where(first, 0.0, kt_ref[...])
    prev_v = jnp.where(first, 0.0, vt_ref[...])
    kext = jnp.concatenate([prev_k[1:], kb], axis=0)
    vext = jnp.concatenate([prev_v[1:], vb], axis=0)
    kt_ref[...] = kb[T - W:]
    vt_ref[...] = vb[T - W:]

    row = i * T + jax.lax.broadcasted_iota(jnp.int32, (T, H), 0)
    scores = []
    for o in range(W):
        prod = qb * kext[W - 1 - o:W - 1 - o + T]
        s = prod[:, :H]
        for d in range(1, QK):
            s = s + prod[:, d * H:(d + 1) * H]
        s = s * _INV_SQRT_QK
        if o > 0:
            s = jnp.where(row >= o, s, -1e30)
        scores.append(s)
    m = functools.reduce(jnp.maximum, scores)
    exps = [jnp.exp(s - m) for s in scores]
    inv = 1.0 / functools.reduce(lambda a, b: a + b, exps)
    out_bm = None
    for o in range(W):
        p = exps[o] * inv
        pw = jnp.concatenate([p] * VB, axis=1)
        term = pw * vext[W - 1 - o:W - 1 - o + T]
        out_bm = term if out_bm is None else out_bm + term
    res = jnp.dot(out_bm, wo_ref[...], preferred_element_type=jnp.float32)
    out_ref[...] = res + bias_ref[...]


def kernel(hidden_states, Wq, Wk, Wv, Wo, v_emb0, v_emb1):
    b, s, hid = hidden_states.shape
    h2 = hidden_states.reshape(b * s, hid)
    perm = _STD_OF_BM
    wq_bm = Wq[perm].T
    wk_bm = Wk[perm].T
    wv_bm = Wv[perm].T
    wo_eff = ((v_emb1 - v_emb0)[:, None] * Wo.T)[perm]
    bias = (Wo @ v_emb0).reshape(1, hid)
    nb = (b * s) // T

    out = pl.pallas_call(
        _rosa_body,
        grid=(nb,),
        in_specs=[
            pl.BlockSpec((T, hid), lambda i: (i, 0)),
            pl.BlockSpec((hid, H * QK), lambda i: (0, 0)),
            pl.BlockSpec((hid, H * QK), lambda i: (0, 0)),
            pl.BlockSpec((hid, H * VB), lambda i: (0, 0)),
            pl.BlockSpec((H * VB, hid), lambda i: (0, 0)),
            pl.BlockSpec((1, hid), lambda i: (0, 0)),
        ],
        out_specs=pl.BlockSpec((T, hid), lambda i: (i, 0)),
        out_shape=jax.ShapeDtypeStruct((b * s, hid), jnp.float32),
        scratch_shapes=[
            pltpu.VMEM((W, H * QK), jnp.float32),
            pltpu.VMEM((W, H * VB), jnp.float32),
        ],
    )(h2, wq_bm, wk_bm, wv_bm, wo_eff, bias)
    return out.reshape(b, s, hid)


# speedup vs baseline: 23.8111x; 1.5354x over previous
import functools
import math

import jax
import jax.numpy as jnp
import numpy as np
from jax.experimental import pallas as pl
from jax.experimental.pallas import tpu as pltpu

H = 96
QK = 8
VB = 8
W = 8
T = 512

_INV_SQRT_QK = 1.0 / math.sqrt(float(QK))
_G_NP = (np.arange(H * QK)[:, None] // QK == np.arange(H)[None, :]).astype(np.float32)


def _rosa_body(h_ref, wq_ref, wk_ref, wv_ref, wo_ref, bias_ref, g_ref, gt_ref,
               out_ref, kt_ref, vt_ref):
    i = pl.program_id(0)
    h = h_ref[...].astype(jnp.bfloat16)
    q = jnp.dot(h, wq_ref[...], preferred_element_type=jnp.float32)
    k = jnp.dot(h, wk_ref[...], preferred_element_type=jnp.float32)
    v = jnp.dot(h, wv_ref[...], preferred_element_type=jnp.float32)
    qb = jnp.tanh(q).astype(jnp.bfloat16)
    kb = jnp.tanh(k).astype(jnp.bfloat16)
    vb = jax.nn.sigmoid(v).astype(jnp.bfloat16)

    zero = jnp.zeros((), jnp.bfloat16)
    prev_k = jnp.where(i == 0, zero, kt_ref[...])
    prev_v = jnp.where(i == 0, zero, vt_ref[...])
    kext = jnp.concatenate([prev_k, kb], axis=0)
    vext = jnp.concatenate([prev_v, vb], axis=0)
    kt_ref[...] = kb[T - W:]
    vt_ref[...] = vb[T - W:]

    row = i * T + jax.lax.broadcasted_iota(jnp.int32, (T, H), 0)
    scores = []
    for o in range(W):
        prod = qb * kext[W - o:W - o + T]
        s = jnp.dot(prod, g_ref[...],
                    preferred_element_type=jnp.float32) * _INV_SQRT_QK
        if o > 0:
            s = jnp.where(row >= o, s, -1e30)
        scores.append(s)
    m = functools.reduce(jnp.maximum, scores)
    exps = [jnp.exp(s - m) for s in scores]
    inv = 1.0 / functools.reduce(lambda a, b: a + b, exps)
    acc = None
    for o in range(W):
        p = (exps[o] * inv).astype(jnp.bfloat16)
        pw = jnp.dot(p, gt_ref[...],
                     preferred_element_type=jnp.float32).astype(jnp.bfloat16)
        term = pw * vext[W - o:W - o + T]
        acc = term if acc is None else acc + term
    res = jnp.dot(acc, wo_ref[...], preferred_element_type=jnp.float32)
    out_ref[...] = res + bias_ref[...]


def kernel(hidden_states, Wq, Wk, Wv, Wo, v_emb0, v_emb1):
    b, s, hid = hidden_states.shape
    h2 = hidden_states.reshape(b * s, hid)
    wq = Wq.T.astype(jnp.bfloat16)
    wk = Wk.T.astype(jnp.bfloat16)
    wv = Wv.T.astype(jnp.bfloat16)
    wo = ((v_emb1 - v_emb0)[:, None] * Wo.T).astype(jnp.bfloat16)
    bias = (Wo @ v_emb0).reshape(1, hid)
    g = jnp.asarray(_G_NP, dtype=jnp.bfloat16)
    gt = jnp.asarray(_G_NP.T, dtype=jnp.bfloat16)
    nb = (b * s) // T

    out = pl.pallas_call(
        _rosa_body,
        grid=(nb,),
        in_specs=[
            pl.BlockSpec((T, hid), lambda i: (i, 0)),
            pl.BlockSpec((hid, H * QK), lambda i: (0, 0)),
            pl.BlockSpec((hid, H * QK), lambda i: (0, 0)),
            pl.BlockSpec((hid, H * VB), lambda i: (0, 0)),
            pl.BlockSpec((H * VB, hid), lambda i: (0, 0)),
            pl.BlockSpec((1, hid), lambda i: (0, 0)),
            pl.BlockSpec((H * QK, H), lambda i: (0, 0)),
            pl.BlockSpec((H, H * VB), lambda i: (0, 0)),
        ],
        out_specs=pl.BlockSpec((T, hid), lambda i: (i, 0)),
        out_shape=jax.ShapeDtypeStruct((b * s, hid), jnp.float32),
        scratch_shapes=[
            pltpu.VMEM((W, H * QK), jnp.bfloat16),
            pltpu.VMEM((W, H * VB), jnp.bfloat16),
        ],
    )(h2, wq, wk, wv, wo, bias, g, gt)
    return out.reshape(b, s, hid)


# speedup vs baseline: 24.2439x; 1.0182x over previous
import functools
import math

import jax
import jax.numpy as jnp
import numpy as np
from jax.experimental import pallas as pl
from jax.experimental.pallas import tpu as pltpu

H = 96
QK = 8
VB = 8
W = 8
T = 512

_INV_SQRT_QK = 1.0 / math.sqrt(float(QK))
_STD_OF_BM = np.arange(H * QK).reshape(H, QK).T.reshape(-1)
_G_BM = np.tile(np.eye(H, dtype=np.float32), (QK, 1))


def _rosa_body(h_ref, wq_ref, wk_ref, wv_ref, wo_ref, bias_ref, g_ref,
               out_ref, kext_ref, vext_ref):
    i = pl.program_id(0)
    h = h_ref[...].astype(jnp.bfloat16)
    q = jnp.dot(h, wq_ref[...], preferred_element_type=jnp.float32)
    k = jnp.dot(h, wk_ref[...], preferred_element_type=jnp.float32)
    v = jnp.dot(h, wv_ref[...], preferred_element_type=jnp.float32)
    qb = jnp.tanh(q).astype(jnp.bfloat16)
    kb = jnp.tanh(k).astype(jnp.bfloat16)
    vb = jax.nn.sigmoid(v).astype(jnp.bfloat16)

    @pl.when(i == 0)
    def _init_halo():
        kext_ref[0:W] = jnp.zeros((W, H * QK), jnp.bfloat16)
        vext_ref[0:W] = jnp.zeros((W, H * VB), jnp.bfloat16)

    @pl.when(i > 0)
    def _carry_halo():
        kext_ref[0:W] = kext_ref[T:T + W]
        vext_ref[0:W] = vext_ref[T:T + W]

    kext_ref[W:] = kb
    vext_ref[W:] = vb

    row = i * T + jax.lax.broadcasted_iota(jnp.int32, (T, H), 0)
    scores = []
    for o in range(W):
        prod = qb * (kb if o == 0 else kext_ref[W - o:W - o + T])
        s = jnp.dot(prod, g_ref[...],
                    preferred_element_type=jnp.float32) * _INV_SQRT_QK
        if o > 0:
            s = jnp.where(row >= o, s, -1e30)
        scores.append(s)
    m = functools.reduce(jnp.maximum, scores)
    exps = [jnp.exp(s - m) for s in scores]
    inv = 1.0 / functools.reduce(lambda a, b: a + b, exps)
    acc = None
    for o in range(W):
        p = (exps[o] * inv).astype(jnp.bfloat16)
        pw = jnp.concatenate([p] * VB, axis=1)
        term = pw * (vb if o == 0 else vext_ref[W - o:W - o + T])
        acc = term if acc is None else acc + term
    res = jnp.dot(acc, wo_ref[...], preferred_element_type=jnp.float32)
    out_ref[...] = res + bias_ref[...]


def kernel(hidden_states, Wq, Wk, Wv, Wo, v_emb0, v_emb1):
    b, s, hid = hidden_states.shape
    h2 = hidden_states.reshape(b * s, hid)
    perm = _STD_OF_BM
    wq = Wq[perm].T.astype(jnp.bfloat16)
    wk = Wk[perm].T.astype(jnp.bfloat16)
    wv = Wv[perm].T.astype(jnp.bfloat16)
    wo = (((v_emb1 - v_emb0)[:, None] * Wo.T)[perm]).astype(jnp.bfloat16)
    bias = (Wo @ v_emb0).reshape(1, hid)
    g = jnp.asarray(_G_BM, dtype=jnp.bfloat16)
    nb = (b * s) // T

    out = pl.pallas_call(
        _rosa_body,
        grid=(nb,),
        in_specs=[
            pl.BlockSpec((T, hid), lambda i: (i, 0)),
            pl.BlockSpec((hid, H * QK), lambda i: (0, 0)),
            pl.BlockSpec((hid, H * QK), lambda i: (0, 0)),
            pl.BlockSpec((hid, H * VB), lambda i: (0, 0)),
            pl.BlockSpec((H * VB, hid), lambda i: (0, 0)),
            pl.BlockSpec((1, hid), lambda i: (0, 0)),
            pl.BlockSpec((H * QK, H), lambda i: (0, 0)),
        ],
        out_specs=pl.BlockSpec((T, hid), lambda i: (i, 0)),
        out_shape=jax.ShapeDtypeStruct((b * s, hid), jnp.float32),
        scratch_shapes=[
            pltpu.VMEM((T + W, H * QK), jnp.bfloat16),
            pltpu.VMEM((T + W, H * VB), jnp.bfloat16),
        ],
    )(h2, wq, wk, wv, wo, bias, g)
    return out.reshape(b, s, hid)
